# baseline (speedup 1.0000x reference)
import jax
import jax.numpy as jnp
from jax.experimental import pallas as pl
from jax.experimental.pallas import tpu as pltpu

_BM1 = 200
_BM23 = 400


_BMX = 2000


def _fused_kernel(x_any, adj_any, w0_ref, b0_ref, w1_ref, b1_ref, wo_ref,
                  bo_ref, out_any, adjb_any, g0_ref, g1_ref, g2_ref):
    n = adj_any.shape[0]
    d_in = x_any.shape[1]

    w0b = w0_ref[...].astype(jnp.bfloat16)

    def body0(x_blk):
        i = pl.program_id(0)
        g0_ref[pl.ds(i * _BMX, _BMX), :] = jnp.dot(
            x_blk[...].astype(jnp.bfloat16), w0b,
            preferred_element_type=jnp.float32).astype(jnp.bfloat16)

    pltpu.emit_pipeline(
        body0, grid=(n // _BMX,),
        in_specs=[pl.BlockSpec((_BMX, d_in), lambda i: (i, 0))],
    )(x_any)

    nb1 = n // _BM1
    w1b = w1_ref[...].astype(jnp.bfloat16)

    def body1(adj_blk, adjb_blk):
        i = pl.program_id(0)
        a = adj_blk[...].astype(jnp.bfloat16)
        adjb_blk[...] = a
        acc = jnp.dot(a, g0_ref[...], preferred_element_type=jnp.float32)
        h1 = jnp.maximum(acc + b0_ref[...], 0.0)
        g1_ref[pl.ds(i * _BM1, _BM1), :] = jnp.dot(
            h1.astype(jnp.bfloat16), w1b,
            preferred_element_type=jnp.float32).astype(jnp.bfloat16)

    pltpu.emit_pipeline(
        body1, grid=(nb1,),
        in_specs=[pl.BlockSpec((_BM1, n), lambda i: (i, 0),
                               pipeline_mode=pl.Buffered(buffer_count=4))],
        out_specs=[pl.BlockSpec((_BM1, n), lambda i: (i, 0))],
    )(adj_any, adjb_any)

    nb = n // _BM23
    stream_spec = pl.BlockSpec((_BM23, n), lambda i: (i, 0),
                               pipeline_mode=pl.Buffered(buffer_count=6))

    def body2(adjb_blk):
        i = pl.program_id(0)
        acc = jnp.dot(adjb_blk[...], g1_ref[...],
                      preferred_element_type=jnp.float32)
        h2 = jnp.maximum(acc + b1_ref[...], 0.0)
        g2_ref[pl.ds(i * _BM23, _BM23), :] = jnp.sum(
            h2 * wo_ref[...], axis=1, keepdims=True).astype(jnp.bfloat16)

    pltpu.emit_pipeline(
        body2, grid=(nb,), in_specs=[stream_spec],
    )(adjb_any)

    def body3(adjb_blk, out_blk):
        acc = jnp.dot(adjb_blk[...], g2_ref[...],
                      preferred_element_type=jnp.float32)
        out_blk[...] = jax.nn.softplus(acc + bo_ref[...])

    pltpu.emit_pipeline(
        body3, grid=(nb,), in_specs=[stream_spec],
        out_specs=[pl.BlockSpec((_BM23, 1), lambda i: (i, 0))],
    )(adjb_any, out_any)


def kernel(x, adj, W0, b0, W1, b1, Wo, bo):
    n, d_in = x.shape
    d_h = W0.shape[1]

    vmem = pl.BlockSpec(memory_space=pltpu.MemorySpace.VMEM)
    out, _ = pl.pallas_call(
        _fused_kernel,
        in_specs=[
            pl.BlockSpec(memory_space=pl.ANY),
            pl.BlockSpec(memory_space=pl.ANY),
            vmem, vmem, vmem, vmem, vmem, vmem,
        ],
        out_specs=[
            pl.BlockSpec(memory_space=pl.ANY),
            pl.BlockSpec(memory_space=pl.ANY),
        ],
        out_shape=[
            jax.ShapeDtypeStruct((n, 1), jnp.float32),
            jax.ShapeDtypeStruct((n, n), jnp.bfloat16),
        ],
        scratch_shapes=[
            pltpu.VMEM((n, d_h), jnp.bfloat16),
            pltpu.VMEM((n, d_h), jnp.bfloat16),
            pltpu.VMEM((n, 1), jnp.bfloat16),
        ],
    )(x, adj, W0, b0.reshape(1, d_h), W1, b1.reshape(1, d_h),
      Wo.reshape(1, d_h), bo.reshape(1, 1))

    return out


# speedup vs baseline: 1.0051x; 1.0051x over previous
import jax
import jax.numpy as jnp
from jax.experimental import pallas as pl
from jax.experimental.pallas import tpu as pltpu

_BM1 = 200
_BM23 = 800


_BMX = 2000


def _fused_kernel(x_any, adj_any, w0_ref, b0_ref, w1_ref, b1_ref, wo_ref,
                  bo_ref, out_any, adjb_any, g0_ref, g1_ref, g2_ref):
    n = adj_any.shape[0]
    d_in = x_any.shape[1]

    w0b = w0_ref[...].astype(jnp.bfloat16)

    def body0(x_blk):
        i = pl.program_id(0)
        g0_ref[pl.ds(i * _BMX, _BMX), :] = jnp.dot(
            x_blk[...].astype(jnp.bfloat16), w0b,
            preferred_element_type=jnp.float32).astype(jnp.bfloat16)

    pltpu.emit_pipeline(
        body0, grid=(n // _BMX,),
        in_specs=[pl.BlockSpec((_BMX, d_in), lambda i: (i, 0))],
    )(x_any)

    nb1 = n // _BM1
    w1b = w1_ref[...].astype(jnp.bfloat16)

    def body1(adj_blk, adjb_blk):
        i = pl.program_id(0)
        a = adj_blk[...].astype(jnp.bfloat16)
        adjb_blk[...] = a
        acc = jnp.dot(a, g0_ref[...], preferred_element_type=jnp.float32)
        h1 = jnp.maximum(acc + b0_ref[...], 0.0)
        g1_ref[pl.ds(i * _BM1, _BM1), :] = jnp.dot(
            h1.astype(jnp.bfloat16), w1b,
            preferred_element_type=jnp.float32).astype(jnp.bfloat16)

    pltpu.emit_pipeline(
        body1, grid=(nb1,),
        in_specs=[pl.BlockSpec((_BM1, n), lambda i: (i, 0),
                               pipeline_mode=pl.Buffered(buffer_count=4))],
        out_specs=[pl.BlockSpec((_BM1, n), lambda i: (i, 0))],
    )(adj_any, adjb_any)

    nb = n // _BM23
    stream_spec = pl.BlockSpec((_BM23, n), lambda i: (i, 0),
                               pipeline_mode=pl.Buffered(buffer_count=3))

    def body2(adjb_blk):
        i = pl.program_id(0)
        acc = jnp.dot(adjb_blk[...], g1_ref[...],
                      preferred_element_type=jnp.float32)
        h2 = jnp.maximum(acc + b1_ref[...], 0.0)
        g2_ref[pl.ds(i * _BM23, _BM23), :] = jnp.sum(
            h2 * wo_ref[...], axis=1, keepdims=True).astype(jnp.bfloat16)

    pltpu.emit_pipeline(
        body2, grid=(nb,), in_specs=[stream_spec],
    )(adjb_any)

    def body3(adjb_blk, out_blk):
        acc = jnp.dot(adjb_blk[...], g2_ref[...],
                      preferred_element_type=jnp.float32)
        out_blk[...] = jax.nn.softplus(acc + bo_ref[...])

    pltpu.emit_pipeline(
        body3, grid=(nb,), in_specs=[stream_spec],
        out_specs=[pl.BlockSpec((_BM23, 1), lambda i: (i, 0))],
    )(adjb_any, out_any)


def kernel(x, adj, W0, b0, W1, b1, Wo, bo):
    n, d_in = x.shape
    d_h = W0.shape[1]

    vmem = pl.BlockSpec(memory_space=pltpu.MemorySpace.VMEM)
    out, _ = pl.pallas_call(
        _fused_kernel,
        in_specs=[
            pl.BlockSpec(memory_space=pl.ANY),
            pl.BlockSpec(memory_space=pl.ANY),
            vmem, vmem, vmem, vmem, vmem, vmem,
        ],
        out_specs=[
            pl.BlockSpec(memory_space=pl.ANY),
            pl.BlockSpec(memory_space=pl.ANY),
        ],
        out_shape=[
            jax.ShapeDtypeStruct((n, 1), jnp.float32),
            jax.ShapeDtypeStruct((n, n), jnp.bfloat16),
        ],
        scratch_shapes=[
            pltpu.VMEM((n, d_h), jnp.bfloat16),
            pltpu.VMEM((n, d_h), jnp.bfloat16),
            pltpu.VMEM((n, 1), jnp.bfloat16),
        ],
    )(x, adj, W0, b0.reshape(1, d_h), W1, b1.reshape(1, d_h),
      Wo.reshape(1, d_h), bo.reshape(1, 1))

    return out


# speedup vs baseline: 1.0128x; 1.0077x over previous
import jax
import jax.numpy as jnp
from jax.experimental import pallas as pl
from jax.experimental.pallas import tpu as pltpu

_BM1 = 200
_BM23 = 400


_BMX = 2000


def _fused_kernel(x_any, adj_any, w0_ref, b0_ref, w1_ref, b1_ref, wo_ref,
                  bo_ref, out_any, adjb_any, g0_ref, g1_ref, g2_ref, g3_ref,
                  dma_sem):
    n = adj_any.shape[0]
    d_in = x_any.shape[1]

    w0b = w0_ref[...].astype(jnp.bfloat16)

    def body0(x_blk):
        i = pl.program_id(0)
        g0_ref[pl.ds(i * _BMX, _BMX), :] = jnp.dot(
            x_blk[...].astype(jnp.bfloat16), w0b,
            preferred_element_type=jnp.float32).astype(jnp.bfloat16)

    pltpu.emit_pipeline(
        body0, grid=(n // _BMX,),
        in_specs=[pl.BlockSpec((_BMX, d_in), lambda i: (i, 0))],
    )(x_any)

    nb1 = n // _BM1
    w1b = w1_ref[...].astype(jnp.bfloat16)

    def body1(adj_blk, adjb_blk):
        i = pl.program_id(0)
        a = adj_blk[...].astype(jnp.bfloat16)
        adjb_blk[...] = a
        acc = jnp.dot(a, g0_ref[...], preferred_element_type=jnp.float32)
        h1 = jnp.maximum(acc + b0_ref[...], 0.0)
        g1_ref[pl.ds(i * _BM1, _BM1), :] = jnp.dot(
            h1.astype(jnp.bfloat16), w1b,
            preferred_element_type=jnp.float32).astype(jnp.bfloat16)

    pltpu.emit_pipeline(
        body1, grid=(nb1,),
        in_specs=[pl.BlockSpec((_BM1, n), lambda i: (i, 0),
                               pipeline_mode=pl.Buffered(buffer_count=4))],
        out_specs=[pl.BlockSpec((_BM1, n), lambda i: (i, 0))],
    )(adj_any, adjb_any)

    nb = n // _BM23
    stream_spec = pl.BlockSpec((_BM23, n), lambda i: (i % nb, 0),
                               pipeline_mode=pl.Buffered(buffer_count=5))

    def body23(adjb_blk):
        i = pl.program_id(0)

        @pl.when(i < nb)
        def _stage2():
            acc = jnp.dot(adjb_blk[...], g1_ref[...],
                          preferred_element_type=jnp.float32)
            h2 = jnp.maximum(acc + b1_ref[...], 0.0)
            g2_ref[pl.ds(i * _BM23, _BM23), :] = jnp.sum(
                h2 * wo_ref[...], axis=1, keepdims=True).astype(jnp.bfloat16)

        @pl.when(i >= nb)
        def _stage3():
            acc = jnp.dot(adjb_blk[...], g2_ref[...],
                          preferred_element_type=jnp.float32)
            g3_ref[pl.ds((i - nb) * _BM23, _BM23), :] = jax.nn.softplus(
                acc + bo_ref[...])

    pltpu.emit_pipeline(
        body23, grid=(2 * nb,), in_specs=[stream_spec],
    )(adjb_any)

    copy = pltpu.make_async_copy(g3_ref, out_any, dma_sem)
    copy.start()
    copy.wait()


def kernel(x, adj, W0, b0, W1, b1, Wo, bo):
    n, d_in = x.shape
    d_h = W0.shape[1]

    vmem = pl.BlockSpec(memory_space=pltpu.MemorySpace.VMEM)
    out, _ = pl.pallas_call(
        _fused_kernel,
        in_specs=[
            pl.BlockSpec(memory_space=pl.ANY),
            pl.BlockSpec(memory_space=pl.ANY),
            vmem, vmem, vmem, vmem, vmem, vmem,
        ],
        out_specs=[
            pl.BlockSpec(memory_space=pl.ANY),
            pl.BlockSpec(memory_space=pl.ANY),
        ],
        out_shape=[
            jax.ShapeDtypeStruct((n, 1), jnp.float32),
            jax.ShapeDtypeStruct((n, n), jnp.bfloat16),
        ],
        scratch_shapes=[
            pltpu.VMEM((n, d_h), jnp.bfloat16),
            pltpu.VMEM((n, d_h), jnp.bfloat16),
            pltpu.VMEM((n, 1), jnp.bfloat16),
            pltpu.VMEM((n, 1), jnp.float32),
            pltpu.SemaphoreType.DMA,
        ],
    )(x, adj, W0, b0.reshape(1, d_h), W1, b1.reshape(1, d_h),
      Wo.reshape(1, d_h), bo.reshape(1, 1))

    return out
